# Initial kernel scaffold; baseline (speedup 1.0000x reference)
import jax
import jax.numpy as jnp
from jax.experimental import pallas as pl


def kernel(x, W0s, W1s, W2s, W0_L, B1_rows, B1_cols, B1_vals, B2_rows, B2_cols, B2_vals):
    raise NotImplementedError("write your pallas kernel here")


# speedup vs baseline: 1.0398x; 1.0398x over previous
import jax
import jax.numpy as jnp
from jax.experimental import pallas as pl

_E = 160000
_F = 128
_BLK = 1280
_G = _E // _BLK


def _mm3_body(x_ref, w0_ref, w1_ref, w2_ref, o0, o1, o2):
    xb = x_ref[...]
    o0[...] = jnp.dot(xb, w0_ref[...], preferred_element_type=jnp.float32)
    o1[...] = jnp.dot(xb, w1_ref[...], preferred_element_type=jnp.float32)
    o2[...] = jnp.dot(xb, w2_ref[...], preferred_element_type=jnp.float32)


def _relu_mm3_body(a_ref, b_ref, c_ref, w0_ref, w1_ref, w2_ref, o0, o1, o2):
    h = jnp.maximum(a_ref[...] + b_ref[...] + c_ref[...], 0.0)
    o0[...] = jnp.dot(h, w0_ref[...], preferred_element_type=jnp.float32)
    o1[...] = jnp.dot(h, w1_ref[...], preferred_element_type=jnp.float32)
    o2[...] = jnp.dot(h, w2_ref[...], preferred_element_type=jnp.float32)


def _relu_mm1_body(a_ref, b_ref, c_ref, w_ref, o0):
    h = jnp.maximum(a_ref[...] + b_ref[...] + c_ref[...], 0.0)
    o0[...] = jnp.dot(h, w_ref[...], preferred_element_type=jnp.float32)


def _row_spec(width):
    return pl.BlockSpec((_BLK, width), lambda i: (i, 0))


def _w_spec(width):
    return pl.BlockSpec((_F, width), lambda i: (0, 0))


_mm3 = pl.pallas_call(
    _mm3_body,
    grid=(_G,),
    in_specs=[_row_spec(_F), _w_spec(_F), _w_spec(_F), _w_spec(_F)],
    out_specs=[_row_spec(_F)] * 3,
    out_shape=[jax.ShapeDtypeStruct((_E, _F), jnp.float32)] * 3,
)

_relu_mm3 = pl.pallas_call(
    _relu_mm3_body,
    grid=(_G,),
    in_specs=[_row_spec(_F)] * 3 + [_w_spec(_F)] * 3,
    out_specs=[_row_spec(_F)] * 3,
    out_shape=[jax.ShapeDtypeStruct((_E, _F), jnp.float32)] * 3,
)

_relu_mm1 = pl.pallas_call(
    _relu_mm1_body,
    grid=(_G,),
    in_specs=[_row_spec(_F)] * 3 + [_w_spec(8)],
    out_specs=_row_spec(8),
    out_shape=jax.ShapeDtypeStruct((_E, 8), jnp.float32),
)


def _seg(rows, cols, vals, src, n_out):
    return jax.ops.segment_sum(vals[:, None] * src[cols], rows, num_segments=n_out)


def kernel(x, W0s, W1s, W2s, W0_L, B1_rows, B1_cols, B1_vals, B2_rows, B2_cols, B2_vals):
    p0, d1, p2 = _mm3(x, W0s[0], W1s[0], W2s[0])
    for i in range(2):
        t = _seg(B2_cols, B2_rows, B2_vals, p2, 80000)
        d2 = _seg(B2_rows, B2_cols, B2_vals, t, _E)
        n = _seg(B1_rows, B1_cols, B1_vals, p0, 10000)
        d0 = _seg(B1_cols, B1_rows, B1_vals, n, _E)
        if i == 0:
            p0, d1, p2 = _relu_mm3(d0, d1, d2, W0s[1], W1s[1], W2s[1])
        else:
            wl = jnp.pad(W0_L, ((0, 0), (0, 7)))
            hw = _relu_mm1(d0, d1, d2, wl)
    out = _seg(B1_rows, B1_cols, B1_vals, hw, 10000)
    return out[:, :1]


# speedup vs baseline: 1.4417x; 1.3866x over previous
import functools

import jax
import jax.numpy as jnp
from jax import lax
from jax.experimental import pallas as pl
from jax.experimental.pallas import tpu as pltpu
from jax.experimental.pallas import tpu_sc as plsc

_E = 160000
_F = 128
_CH = 8192
_CHN = 6144
_E_PAD = 20 * _CH
_T_PAD = 10 * _CH
_N_PAD = 2 * _CHN
_BLK = 1280
_TILE = 128


def _mm3_body(x_ref, w0_ref, w1_ref, w2_ref, o0, o1, o2):
    xb = x_ref[...]
    o0[...] = jnp.dot(xb, w0_ref[...], preferred_element_type=jnp.float32)
    o1[...] = jnp.dot(xb, w1_ref[...], preferred_element_type=jnp.float32)
    o2[...] = jnp.dot(xb, w2_ref[...], preferred_element_type=jnp.float32)


def _relu_mm3_body(a_ref, b_ref, c_ref, w0_ref, w1_ref, w2_ref, o0, o1, o2):
    h = jnp.maximum(a_ref[...] + b_ref[...] + c_ref[...], 0.0)
    o0[...] = jnp.dot(h, w0_ref[...], preferred_element_type=jnp.float32)
    o1[...] = jnp.dot(h, w1_ref[...], preferred_element_type=jnp.float32)
    o2[...] = jnp.dot(h, w2_ref[...], preferred_element_type=jnp.float32)


def _relu_mm1_body(a_ref, b_ref, c_ref, w_ref, o0):
    h = jnp.maximum(a_ref[...] + b_ref[...] + c_ref[...], 0.0)
    o0[...] = jnp.dot(h, w_ref[...], preferred_element_type=jnp.float32)


def _row_spec(width):
    return pl.BlockSpec((_BLK, width), lambda i: (i, 0))


def _w_spec(width):
    return pl.BlockSpec((_F, width), lambda i: (0, 0))


_G = _E_PAD // _BLK

_mm3 = pl.pallas_call(
    _mm3_body,
    grid=(_G,),
    in_specs=[_row_spec(_F), _w_spec(_F), _w_spec(_F), _w_spec(_F)],
    out_specs=[_row_spec(_F)] * 3,
    out_shape=[jax.ShapeDtypeStruct((_E_PAD, _F), jnp.float32)] * 3,
)

_relu_mm3 = pl.pallas_call(
    _relu_mm3_body,
    grid=(_G,),
    in_specs=[_row_spec(_F)] * 3 + [_w_spec(_F)] * 3,
    out_specs=[_row_spec(_F)] * 3,
    out_shape=[jax.ShapeDtypeStruct((_E_PAD, _F), jnp.float32)] * 3,
)

_relu_mm1 = pl.pallas_call(
    _relu_mm1_body,
    grid=(_G,),
    in_specs=[_row_spec(_F)] * 3 + [_w_spec(_F)],
    out_specs=_row_spec(_F),
    out_shape=jax.ShapeDtypeStruct((_E_PAD, _F), jnp.float32),
)


def _prep(tgt, srcidx, vals, nchunk, ch):
    nnz = tgt.shape[0]
    nnzp = nnz + _TILE * nchunk
    order = jnp.argsort(tgt)
    tgt_s = tgt[order]
    src_s = srcidx[order]
    val_s = vals[order]
    edges = (jnp.arange(nchunk + 1, dtype=jnp.int32) * ch).astype(tgt_s.dtype)
    bounds = jnp.searchsorted(tgt_s, edges).astype(jnp.int32)
    cnt = bounds[1:] - bounds[:-1]
    tiles = (cnt + _TILE - 1) // _TILE
    padlen = tiles * _TILE
    pend = jnp.cumsum(padlen)
    pstart = (pend - padlen).astype(jnp.int32)
    j = jnp.arange(nnzp, dtype=jnp.int32)
    c_of = jnp.clip(jnp.searchsorted(pend, j, side="right"), 0, nchunk - 1)
    within = j - pstart[c_of]
    valid = within < cnt[c_of]
    spos = jnp.clip(bounds[:-1][c_of] + within, 0, nnz - 1)
    psrc = jnp.where(valid, src_s[spos], 0).astype(jnp.int32)
    ptgt = jnp.where(valid, tgt_s[spos], c_of * ch).astype(jnp.int32)
    pval = jnp.where(valid, val_s[spos], 0.0).astype(jnp.float32)
    pvalr = jnp.broadcast_to(pval[:, None], (nnzp, 16)).copy()
    meta_t = jnp.broadcast_to(tiles[:, None], (nchunk, 16)).reshape(-1)
    meta_s = jnp.broadcast_to((pstart // _TILE)[:, None], (nchunk, 16)).reshape(-1)
    return psrc, ptgt, pvalr, meta_t, meta_s


@functools.lru_cache(maxsize=None)
def _make_spmm(nnzp, nchunk, ch, f):
    del nnzp
    out_rows = nchunk * ch
    sub_rows = ch // 16
    nzcopy = sub_rows // _TILE
    mesh = plsc.VectorSubcoreMesh(core_axis_name="c", subcore_axis_name="s")

    @functools.partial(
        pl.kernel,
        mesh=mesh,
        out_type=jax.ShapeDtypeStruct((out_rows, f), jnp.float32),
        scratch_types=[
            pltpu.VMEM((_TILE,), jnp.int32),
            pltpu.VMEM((_TILE,), jnp.int32),
            pltpu.VMEM((_TILE,), jnp.int32),
            pltpu.VMEM((_TILE, 16), jnp.float32),
            pltpu.VMEM((_TILE, f), jnp.float32),
            pltpu.VMEM((_TILE, f), jnp.float32),
            pltpu.VMEM((nchunk * 16,), jnp.int32),
            pltpu.VMEM((nchunk * 16,), jnp.int32),
            pltpu.VMEM_SHARED((ch, f), jnp.float32),
            pltpu.SemaphoreType.DMA,
        ],
    )
    def spmm(src_hbm, psrc_hbm, ptgt_hbm, pvalr_hbm, mt_hbm, ms_hbm, z_hbm,
             out_hbm, idx_v, tgtg_v, tgtl_v, val_v, rows_v, zero_v, mt_v,
             ms_v, acc_sh, sem):
        cid = lax.axis_index("c")
        sid = lax.axis_index("s")
        pltpu.sync_copy(mt_hbm, mt_v)
        pltpu.sync_copy(ms_hbm, ms_v)
        pltpu.sync_copy(z_hbm, zero_v)

        def chunk_body(ci, _):
            c = 2 * ci + cid
            for z in range(nzcopy):
                r0 = sid * sub_rows + z * _TILE
                pltpu.sync_copy(zero_v, acc_sh.at[pl.ds(r0, _TILE), :])
            plsc.subcore_barrier()
            t_c = mt_v[pl.ds(c * 16, 16)][0]
            s_c = ms_v[pl.ds(c * 16, 16)][0]
            my_tiles = (t_c - sid + 15) // 16
            base_l = c * ch

            def tile_body(it, _):
                off = (s_c + (sid + it * 16)) * _TILE
                pltpu.sync_copy(psrc_hbm.at[pl.ds(off, _TILE)], idx_v)
                pltpu.sync_copy(ptgt_hbm.at[pl.ds(off, _TILE)], tgtg_v)
                pltpu.sync_copy(pvalr_hbm.at[pl.ds(off, _TILE), :], val_v)
                pltpu.async_copy(src_hbm.at[idx_v], rows_v, sem).wait()

                def grp(g, _):
                    tg = tgtg_v[pl.ds(g * 16, 16)]
                    tgtl_v[pl.ds(g * 16, 16)] = tg - base_l
                    return 0

                lax.fori_loop(0, _TILE // 16, grp, 0)

                def ent(k, _):
                    vv = val_v[k, :]
                    for jj in range(f // 16):
                        sl = pl.ds(jj * 16, 16)
                        rows_v[k, sl] = rows_v[k, sl] * vv
                    return 0

                lax.fori_loop(0, _TILE, ent, 0)
                pltpu.sync_copy(rows_v, acc_sh.at[tgtl_v], add=True)
                return 0

            lax.fori_loop(0, my_tiles, tile_body, 0)
            plsc.subcore_barrier()
            for z in range(nzcopy):
                r0 = sid * sub_rows + z * _TILE
                pltpu.sync_copy(acc_sh.at[pl.ds(r0, _TILE), :], rows_v)
                pltpu.sync_copy(rows_v, out_hbm.at[pl.ds(base_l + r0, _TILE), :])
            return 0

        lax.fori_loop(0, nchunk // 2, chunk_body, 0)

    return spmm


def kernel(x, W0s, W1s, W2s, W0_L, B1_rows, B1_cols, B1_vals, B2_rows, B2_cols, B2_vals):
    xp = jnp.pad(x, ((0, _E_PAD - _E), (0, 0)))
    z128 = jnp.zeros((_TILE, _F), jnp.float32)

    pb2c = _prep(B2_cols, B2_rows, B2_vals, 10, _CH)
    pb2r = _prep(B2_rows, B2_cols, B2_vals, 20, _CH)
    pb1r = _prep(B1_rows, B1_cols, B1_vals, 2, _CHN)
    pb1c = _prep(B1_cols, B1_rows, B1_vals, 20, _CH)

    spmm_tri = _make_spmm(pb2c[0].shape[0], 10, _CH, _F)
    spmm_e_t = _make_spmm(pb2r[0].shape[0], 20, _CH, _F)
    spmm_n = _make_spmm(pb1r[0].shape[0], 2, _CHN, _F)
    spmm_e_n = _make_spmm(pb1c[0].shape[0], 20, _CH, _F)

    p0, d1, p2 = _mm3(xp, W0s[0], W1s[0], W2s[0])
    hw = None
    for i in range(2):
        t = spmm_tri(p2, *pb2c, z128)
        d2 = spmm_e_t(t, *pb2r, z128)
        nn = spmm_n(p0, *pb1r, z128)
        d0 = spmm_e_n(nn, *pb1c, z128)
        if i == 0:
            p0, d1, p2 = _relu_mm3(d0, d1, d2, W0s[1], W1s[1], W2s[1])
        else:
            hw = _relu_mm1(d0, d1, d2, jnp.pad(W0_L, ((0, 0), (0, 127))))
    res = spmm_n(hw, *pb1r, z128)
    return res[:10000, :1]


# speedup vs baseline: 2.0222x; 1.4026x over previous
import functools

import jax
import jax.numpy as jnp
from jax import lax
from jax.experimental import pallas as pl
from jax.experimental.pallas import tpu as pltpu
from jax.experimental.pallas import tpu_sc as plsc

_E = 160000
_F = 128
_CH = 8192
_CHN = 6144
_E_PAD = 20 * _CH
_T_PAD = 10 * _CH
_N_PAD = 2 * _CHN
_BLK = 1280
_TILE = 128


def _mm3_body(x_ref, w0_ref, w1_ref, w2_ref, o0, o1, o2):
    xb = x_ref[...]
    o0[...] = jnp.dot(xb, w0_ref[...], preferred_element_type=jnp.float32)
    o1[...] = jnp.dot(xb, w1_ref[...], preferred_element_type=jnp.float32)
    o2[...] = jnp.dot(xb, w2_ref[...], preferred_element_type=jnp.float32)


def _relu_mm3_body(a_ref, b_ref, c_ref, w0_ref, w1_ref, w2_ref, o0, o1, o2):
    h = jnp.maximum(a_ref[...] + b_ref[...] + c_ref[...], 0.0)
    o0[...] = jnp.dot(h, w0_ref[...], preferred_element_type=jnp.float32)
    o1[...] = jnp.dot(h, w1_ref[...], preferred_element_type=jnp.float32)
    o2[...] = jnp.dot(h, w2_ref[...], preferred_element_type=jnp.float32)


def _relu_mm1_body(a_ref, b_ref, c_ref, w_ref, o0):
    h = jnp.maximum(a_ref[...] + b_ref[...] + c_ref[...], 0.0)
    o0[...] = jnp.dot(h, w_ref[...], preferred_element_type=jnp.float32)


def _row_spec(width):
    return pl.BlockSpec((_BLK, width), lambda i: (i, 0))


def _w_spec(width):
    return pl.BlockSpec((_F, width), lambda i: (0, 0))


_G = _E_PAD // _BLK

_mm3 = pl.pallas_call(
    _mm3_body,
    grid=(_G,),
    in_specs=[_row_spec(_F), _w_spec(_F), _w_spec(_F), _w_spec(_F)],
    out_specs=[_row_spec(_F)] * 3,
    out_shape=[jax.ShapeDtypeStruct((_E_PAD, _F), jnp.float32)] * 3,
)

_relu_mm3 = pl.pallas_call(
    _relu_mm3_body,
    grid=(_G,),
    in_specs=[_row_spec(_F)] * 3 + [_w_spec(_F)] * 3,
    out_specs=[_row_spec(_F)] * 3,
    out_shape=[jax.ShapeDtypeStruct((_E_PAD, _F), jnp.float32)] * 3,
)

_relu_mm1 = pl.pallas_call(
    _relu_mm1_body,
    grid=(_G,),
    in_specs=[_row_spec(_F)] * 3 + [_w_spec(_F)],
    out_specs=_row_spec(_F),
    out_shape=jax.ShapeDtypeStruct((_E_PAD, _F), jnp.float32),
)


def _prep(tgt, srcidx, vals, nchunk, ch):
    nnz = tgt.shape[0]
    nnzp = nnz + _TILE * nchunk
    order = jnp.argsort(tgt)
    tgt_s = tgt[order]
    src_s = srcidx[order]
    val_s = vals[order]
    edges = (jnp.arange(nchunk + 1, dtype=jnp.int32) * ch).astype(tgt_s.dtype)
    bounds = jnp.searchsorted(tgt_s, edges).astype(jnp.int32)
    cnt = bounds[1:] - bounds[:-1]
    tiles = (cnt + _TILE - 1) // _TILE
    padlen = tiles * _TILE
    pend = jnp.cumsum(padlen)
    pstart = (pend - padlen).astype(jnp.int32)
    j = jnp.arange(nnzp, dtype=jnp.int32)
    c_of = jnp.clip(jnp.searchsorted(pend, j, side="right"), 0, nchunk - 1)
    within = j - pstart[c_of]
    valid = within < cnt[c_of]
    spos = jnp.clip(bounds[:-1][c_of] + within, 0, nnz - 1)
    psrc = jnp.where(valid, src_s[spos], 0).astype(jnp.int32)
    ptgt = jnp.where(valid, tgt_s[spos], c_of * ch).astype(jnp.int32)
    pval = jnp.where(valid, val_s[spos], 0.0).astype(jnp.float32)
    meta_t = jnp.broadcast_to(tiles[:, None], (nchunk, 16)).reshape(-1)
    meta_s = jnp.broadcast_to((pstart // _TILE)[:, None], (nchunk, 16)).reshape(-1)
    return psrc, ptgt, pval, meta_t, meta_s


@functools.lru_cache(maxsize=None)
def _make_spmm(nnzp, nchunk, ch, f):
    del nnzp
    out_rows = nchunk * ch
    sub_rows = ch // 16
    nzcopy = sub_rows // _TILE
    mesh = plsc.VectorSubcoreMesh(core_axis_name="c", subcore_axis_name="s")

    @functools.partial(
        pl.kernel,
        mesh=mesh,
        out_type=jax.ShapeDtypeStruct((out_rows, f), jnp.float32),
        scratch_types=[
            pltpu.VMEM((_TILE,), jnp.int32),
            pltpu.VMEM((_TILE,), jnp.int32),
            pltpu.VMEM((_TILE,), jnp.int32),
            pltpu.VMEM((_TILE,), jnp.int32),
            pltpu.VMEM((_TILE,), jnp.int32),
            pltpu.VMEM((_TILE,), jnp.int32),
            pltpu.VMEM((_TILE,), jnp.float32),
            pltpu.VMEM((_TILE,), jnp.float32),
            pltpu.VMEM((_TILE, f), jnp.float32),
            pltpu.VMEM((_TILE, f), jnp.float32),
            pltpu.VMEM((_TILE, f), jnp.float32),
            pltpu.VMEM((nchunk * 16,), jnp.int32),
            pltpu.VMEM((nchunk * 16,), jnp.int32),
            pltpu.VMEM_SHARED((ch, f), jnp.float32),
            pltpu.SemaphoreType.DMA,
            pltpu.SemaphoreType.DMA,
            pltpu.SemaphoreType.DMA,
            pltpu.SemaphoreType.DMA,
            pltpu.SemaphoreType.DMA,
            pltpu.SemaphoreType.DMA,
            pltpu.SemaphoreType.DMA,
        ],
    )
    def spmm(src_hbm, psrc_hbm, ptgt_hbm, pvalr_hbm, mt_hbm, ms_hbm, z_hbm,
             out_hbm, idx0, idx1, tgt0, tgt1, tl0, tl1, val0, val1, rows0,
             rows1, zero_v, mt_v, ms_v, acc_sh, sa0, sa1, sg0, sg1, ss0,
             ss1, sz):
        idx = (idx0, idx1)
        tgt = (tgt0, tgt1)
        tl = (tl0, tl1)
        val = (val0, val1)
        rows = (rows0, rows1)
        sa = (sa0, sa1)
        sg = (sg0, sg1)
        ss = (ss0, ss1)
        cid = lax.axis_index("c")
        sid = lax.axis_index("s")
        pltpu.sync_copy(mt_hbm, mt_v)
        pltpu.sync_copy(ms_hbm, ms_v)
        pltpu.sync_copy(z_hbm, zero_v)

        def chunk_body(ci, _):
            c = 2 * ci + cid
            for z in range(nzcopy):
                r0 = sid * sub_rows + z * _TILE
                pltpu.async_copy(zero_v, acc_sh.at[pl.ds(r0, _TILE), :], sz)
            for z in range(nzcopy):
                pltpu.make_async_copy(
                    zero_v, acc_sh.at[pl.ds(sid * sub_rows, _TILE), :], sz
                ).wait()
            plsc.subcore_barrier()
            t_c = mt_v[pl.ds(c * 16, 16)][0]
            s_c = ms_v[pl.ds(c * 16, 16)][0]
            my_tiles = (t_c - sid + 15) // 16
            base_l = c * ch

            def issue_in(i, b):
                off = (s_c + (sid + i * 16)) * _TILE
                pltpu.async_copy(psrc_hbm.at[pl.ds(off, _TILE)], idx[b], sa[b])
                pltpu.async_copy(ptgt_hbm.at[pl.ds(off, _TILE)], tgt[b], sa[b])
                pltpu.async_copy(pvalr_hbm.at[pl.ds(off, _TILE)], val[b], sa[b])

            @pl.when(my_tiles > 0)
            def _():
                issue_in(0, 0)

            def pair_body(g, _):
                for b in (0, 1):
                    i = g * 2 + b

                    @pl.when(i < my_tiles)
                    def _(b=b, i=i):
                        pltpu.make_async_copy(
                            psrc_hbm.at[pl.ds(0, _TILE)], idx[b], sa[b]).wait()
                        pltpu.make_async_copy(
                            ptgt_hbm.at[pl.ds(0, _TILE)], tgt[b], sa[b]).wait()
                        pltpu.make_async_copy(
                            pvalr_hbm.at[pl.ds(0, _TILE)], val[b], sa[b]).wait()

                        @pl.when(i >= 2)
                        def _():
                            pltpu.make_async_copy(
                                rows[b], acc_sh.at[tl[b]], ss[b]).wait()

                        gh = pltpu.async_copy(src_hbm.at[idx[b]], rows[b], sg[b])

                        @pl.when(i + 1 < my_tiles)
                        def _():
                            issue_in(i + 1, 1 - b)

                        def grp(gg, _):
                            s = pl.ds(gg * 16, 16)
                            tl[b][s] = tgt[b][s] - base_l
                            return 0

                        lax.fori_loop(0, _TILE // 16, grp, 0)
                        gh.wait()

                        def ent(gg, _):
                            vv = val[b][pl.ds(gg * 16, 16)]
                            for kk in range(16):
                                k = gg * 16 + kk
                                sp = lax.broadcast_in_dim(vv[kk], (16,), ())
                                for jj in range(f // 16):
                                    sl = pl.ds(jj * 16, 16)
                                    rows[b][k, sl] = rows[b][k, sl] * sp
                            return 0

                        lax.fori_loop(0, _TILE // 16, ent, 0)
                        pltpu.async_copy(
                            rows[b], acc_sh.at[tl[b]], ss[b], add=True)
                return 0

            lax.fori_loop(0, (my_tiles + 1) // 2, pair_body, 0)

            @pl.when(my_tiles >= 1)
            def _():
                pltpu.make_async_copy(rows[0], acc_sh.at[tl[0]], ss[0]).wait()

            @pl.when(my_tiles >= 2)
            def _():
                pltpu.make_async_copy(rows[1], acc_sh.at[tl[1]], ss[1]).wait()

            plsc.subcore_barrier()
            for z in range(nzcopy):
                r0 = sid * sub_rows + z * _TILE
                pltpu.async_copy(
                    acc_sh.at[pl.ds(r0, _TILE), :],
                    out_hbm.at[pl.ds(base_l + r0, _TILE), :], sz)
            for z in range(nzcopy):
                pltpu.make_async_copy(
                    acc_sh.at[pl.ds(sid * sub_rows, _TILE), :],
                    out_hbm.at[pl.ds(base_l, _TILE), :], sz).wait()
            return 0

        lax.fori_loop(0, nchunk // 2, chunk_body, 0)

    return spmm


def kernel(x, W0s, W1s, W2s, W0_L, B1_rows, B1_cols, B1_vals, B2_rows, B2_cols, B2_vals):
    xp = jnp.pad(x, ((0, _E_PAD - _E), (0, 0)))
    z128 = jnp.zeros((_TILE, _F), jnp.float32)

    pb2c = _prep(B2_cols, B2_rows, B2_vals, 10, _CH)
    pb2r = _prep(B2_rows, B2_cols, B2_vals, 20, _CH)
    pb1r = _prep(B1_rows, B1_cols, B1_vals, 2, _CHN)
    pb1c = _prep(B1_cols, B1_rows, B1_vals, 20, _CH)

    spmm_tri = _make_spmm(pb2c[0].shape[0], 10, _CH, _F)
    spmm_e_t = _make_spmm(pb2r[0].shape[0], 20, _CH, _F)
    spmm_n = _make_spmm(pb1r[0].shape[0], 2, _CHN, _F)
    spmm_e_n = _make_spmm(pb1c[0].shape[0], 20, _CH, _F)

    p0, d1, p2 = _mm3(xp, W0s[0], W1s[0], W2s[0])
    hw = None
    for i in range(2):
        t = spmm_tri(p2, *pb2c, z128)
        d2 = spmm_e_t(t, *pb2r, z128)
        nn = spmm_n(p0, *pb1r, z128)
        d0 = spmm_e_n(nn, *pb1c, z128)
        if i == 0:
            p0, d1, p2 = _relu_mm3(d0, d1, d2, W0s[1], W1s[1], W2s[1])
        else:
            hw = _relu_mm1(d0, d1, d2, jnp.pad(W0_L, ((0, 0), (0, 127))))
    res = spmm_n(hw, *pb1r, z128)
    return res[:10000, :1]


# speedup vs baseline: 2.0993x; 1.0381x over previous
import functools

import jax
import jax.numpy as jnp
from jax import lax
from jax.experimental import pallas as pl
from jax.experimental.pallas import tpu as pltpu
from jax.experimental.pallas import tpu_sc as plsc

_E = 160000
_F = 128
_CH = 8192
_CHN = 6144
_E_PAD = 20 * _CH
_T_PAD = 10 * _CH
_N_PAD = 2 * _CHN
_BLK = 1280
_TILE = 128


def _mm3_body(x_ref, w0_ref, w1_ref, w2_ref, o0, o1, o2):
    xb = x_ref[...]
    o0[...] = jnp.dot(xb, w0_ref[...], preferred_element_type=jnp.float32)
    o1[...] = jnp.dot(xb, w1_ref[...], preferred_element_type=jnp.float32)
    o2[...] = jnp.dot(xb, w2_ref[...], preferred_element_type=jnp.float32)


def _relu_mm3_body(a_ref, b_ref, c_ref, w0_ref, w1_ref, w2_ref, o0, o1, o2):
    h = jnp.maximum(a_ref[...] + b_ref[...] + c_ref[...], 0.0)
    o0[...] = jnp.dot(h, w0_ref[...], preferred_element_type=jnp.float32)
    o1[...] = jnp.dot(h, w1_ref[...], preferred_element_type=jnp.float32)
    o2[...] = jnp.dot(h, w2_ref[...], preferred_element_type=jnp.float32)


def _relu_mm1_body(a_ref, b_ref, c_ref, w_ref, o0):
    h = jnp.maximum(a_ref[...] + b_ref[...] + c_ref[...], 0.0)
    o0[...] = jnp.dot(h, w_ref[...], preferred_element_type=jnp.float32)


def _row_spec(width):
    return pl.BlockSpec((_BLK, width), lambda i: (i, 0))


def _w_spec(width):
    return pl.BlockSpec((_F, width), lambda i: (0, 0))


_G = _E_PAD // _BLK

_mm3 = pl.pallas_call(
    _mm3_body,
    grid=(_G,),
    in_specs=[_row_spec(_F), _w_spec(_F), _w_spec(_F), _w_spec(_F)],
    out_specs=[_row_spec(_F)] * 3,
    out_shape=[jax.ShapeDtypeStruct((_E_PAD, _F), jnp.float32)] * 3,
)

_relu_mm3 = pl.pallas_call(
    _relu_mm3_body,
    grid=(_G,),
    in_specs=[_row_spec(_F)] * 3 + [_w_spec(_F)] * 3,
    out_specs=[_row_spec(_F)] * 3,
    out_shape=[jax.ShapeDtypeStruct((_E_PAD, _F), jnp.float32)] * 3,
)

_relu_mm1 = pl.pallas_call(
    _relu_mm1_body,
    grid=(_G,),
    in_specs=[_row_spec(_F)] * 3 + [_w_spec(_F)],
    out_specs=_row_spec(_F),
    out_shape=jax.ShapeDtypeStruct((_E_PAD, _F), jnp.float32),
)


def _prep(tgt, srcidx, vals, nchunk, ch):
    nnz = tgt.shape[0]
    nnzp = nnz + _TILE * nchunk
    chunk_id = (tgt // ch).astype(jnp.int32)
    key = jnp.sort(chunk_id * 524288 + jnp.arange(nnz, dtype=jnp.int32))
    order = key % 524288
    tgt_s = tgt[order]
    src_s = srcidx[order]
    val_s = vals[order]
    edges = jnp.arange(nchunk + 1, dtype=jnp.int32) * 524288
    bounds = jnp.searchsorted(key, edges).astype(jnp.int32)
    cnt = bounds[1:] - bounds[:-1]
    tiles = (cnt + _TILE - 1) // _TILE
    padlen = tiles * _TILE
    pend = jnp.cumsum(padlen).astype(jnp.int32)
    pstart = (pend - padlen).astype(jnp.int32)
    j = jnp.arange(nnzp, dtype=jnp.int32)
    c_of = jnp.minimum((j[:, None] >= pend[None, :]).astype(jnp.int32).sum(axis=1),
                       nchunk - 1)
    within = j - pstart[c_of]
    valid = within < cnt[c_of]
    spos = jnp.clip(bounds[:-1][c_of] + within, 0, nnz - 1)
    psrc = jnp.where(valid, src_s[spos], 0).astype(jnp.int32)
    ptgt = jnp.where(valid, tgt_s[spos], c_of * ch).astype(jnp.int32)
    pval = jnp.where(valid, val_s[spos], 0.0).astype(jnp.float32)
    meta_t = jnp.broadcast_to(tiles[:, None], (nchunk, 16)).reshape(-1)
    meta_s = jnp.broadcast_to((pstart // _TILE)[:, None], (nchunk, 16)).reshape(-1)
    return psrc, ptgt, pval, meta_t, meta_s


@functools.lru_cache(maxsize=None)
def _make_spmm(nnzp, nchunk, ch, f):
    del nnzp
    out_rows = nchunk * ch
    sub_rows = ch // 16
    nzcopy = sub_rows // _TILE
    mesh = plsc.VectorSubcoreMesh(core_axis_name="c", subcore_axis_name="s")

    @functools.partial(
        pl.kernel,
        mesh=mesh,
        out_type=jax.ShapeDtypeStruct((out_rows, f), jnp.float32),
        scratch_types=[
            pltpu.VMEM((_TILE,), jnp.int32),
            pltpu.VMEM((_TILE,), jnp.int32),
            pltpu.VMEM((_TILE,), jnp.int32),
            pltpu.VMEM((_TILE,), jnp.int32),
            pltpu.VMEM((_TILE,), jnp.int32),
            pltpu.VMEM((_TILE,), jnp.int32),
            pltpu.VMEM((_TILE,), jnp.float32),
            pltpu.VMEM((_TILE,), jnp.float32),
            pltpu.VMEM((_TILE, f), jnp.float32),
            pltpu.VMEM((_TILE, f), jnp.float32),
            pltpu.VMEM((_TILE, f), jnp.float32),
            pltpu.VMEM((nchunk * 16,), jnp.int32),
            pltpu.VMEM((nchunk * 16,), jnp.int32),
            pltpu.VMEM_SHARED((ch, f), jnp.float32),
            pltpu.SemaphoreType.DMA,
            pltpu.SemaphoreType.DMA,
            pltpu.SemaphoreType.DMA,
            pltpu.SemaphoreType.DMA,
            pltpu.SemaphoreType.DMA,
            pltpu.SemaphoreType.DMA,
            pltpu.SemaphoreType.DMA,
        ],
    )
    def spmm(src_hbm, psrc_hbm, ptgt_hbm, pvalr_hbm, mt_hbm, ms_hbm, z_hbm,
             out_hbm, idx0, idx1, tgt0, tgt1, tl0, tl1, val0, val1, rows0,
             rows1, zero_v, mt_v, ms_v, acc_sh, sa0, sa1, sg0, sg1, ss0,
             ss1, sz):
        idx = (idx0, idx1)
        tgt = (tgt0, tgt1)
        tl = (tl0, tl1)
        val = (val0, val1)
        rows = (rows0, rows1)
        sa = (sa0, sa1)
        sg = (sg0, sg1)
        ss = (ss0, ss1)
        cid = lax.axis_index("c")
        sid = lax.axis_index("s")
        pltpu.sync_copy(mt_hbm, mt_v)
        pltpu.sync_copy(ms_hbm, ms_v)
        pltpu.sync_copy(z_hbm, zero_v)

        def chunk_body(ci, _):
            c = 2 * ci + cid
            for z in range(nzcopy):
                r0 = sid * sub_rows + z * _TILE
                pltpu.async_copy(zero_v, acc_sh.at[pl.ds(r0, _TILE), :], sz)
            for z in range(nzcopy):
                pltpu.make_async_copy(
                    zero_v, acc_sh.at[pl.ds(sid * sub_rows, _TILE), :], sz
                ).wait()
            plsc.subcore_barrier()
            t_c = mt_v[pl.ds(c * 16, 16)][0]
            s_c = ms_v[pl.ds(c * 16, 16)][0]
            my_tiles = (t_c - sid + 15) // 16
            base_l = c * ch

            def issue_in(i, b):
                off = (s_c + (sid + i * 16)) * _TILE
                pltpu.async_copy(psrc_hbm.at[pl.ds(off, _TILE)], idx[b], sa[b])
                pltpu.async_copy(ptgt_hbm.at[pl.ds(off, _TILE)], tgt[b], sa[b])
                pltpu.async_copy(pvalr_hbm.at[pl.ds(off, _TILE)], val[b], sa[b])

            @pl.when(my_tiles > 0)
            def _():
                issue_in(0, 0)

            def pair_body(g, _):
                for b in (0, 1):
                    i = g * 2 + b

                    @pl.when(i < my_tiles)
                    def _(b=b, i=i):
                        pltpu.make_async_copy(
                            psrc_hbm.at[pl.ds(0, _TILE)], idx[b], sa[b]).wait()
                        pltpu.make_async_copy(
                            ptgt_hbm.at[pl.ds(0, _TILE)], tgt[b], sa[b]).wait()
                        pltpu.make_async_copy(
                            pvalr_hbm.at[pl.ds(0, _TILE)], val[b], sa[b]).wait()

                        @pl.when(i >= 2)
                        def _():
                            pltpu.make_async_copy(
                                rows[b], acc_sh.at[tl[b]], ss[b]).wait()

                        gh = pltpu.async_copy(src_hbm.at[idx[b]], rows[b], sg[b])

                        @pl.when(i + 1 < my_tiles)
                        def _():
                            issue_in(i + 1, 1 - b)

                        def grp(gg, _):
                            s = pl.ds(gg * 16, 16)
                            tl[b][s] = tgt[b][s] - base_l
                            return 0

                        lax.fori_loop(0, _TILE // 16, grp, 0)
                        gh.wait()

                        def ent(gg, _):
                            vv = val[b][pl.ds(gg * 16, 16)]
                            for kk in range(16):
                                k = gg * 16 + kk
                                sp = lax.broadcast_in_dim(vv[kk], (16,), ())
                                for jj in range(f // 16):
                                    sl = pl.ds(jj * 16, 16)
                                    rows[b][k, sl] = rows[b][k, sl] * sp
                            return 0

                        lax.fori_loop(0, _TILE // 16, ent, 0)
                        pltpu.async_copy(
                            rows[b], acc_sh.at[tl[b]], ss[b], add=True)
                return 0

            lax.fori_loop(0, (my_tiles + 1) // 2, pair_body, 0)

            @pl.when(my_tiles >= 1)
            def _():
                pltpu.make_async_copy(rows[0], acc_sh.at[tl[0]], ss[0]).wait()

            @pl.when(my_tiles >= 2)
            def _():
                pltpu.make_async_copy(rows[1], acc_sh.at[tl[1]], ss[1]).wait()

            plsc.subcore_barrier()
            for z in range(nzcopy):
                r0 = sid * sub_rows + z * _TILE
                pltpu.async_copy(
                    acc_sh.at[pl.ds(r0, _TILE), :],
                    out_hbm.at[pl.ds(base_l + r0, _TILE), :], sz)
            for z in range(nzcopy):
                pltpu.make_async_copy(
                    acc_sh.at[pl.ds(sid * sub_rows, _TILE), :],
                    out_hbm.at[pl.ds(base_l, _TILE), :], sz).wait()
            return 0

        lax.fori_loop(0, nchunk // 2, chunk_body, 0)

    return spmm


def kernel(x, W0s, W1s, W2s, W0_L, B1_rows, B1_cols, B1_vals, B2_rows, B2_cols, B2_vals):
    xp = jnp.pad(x, ((0, _E_PAD - _E), (0, 0)))
    z128 = jnp.zeros((_TILE, _F), jnp.float32)

    pb2c = _prep(B2_cols, B2_rows, B2_vals, 10, _CH)
    pb2r = _prep(B2_rows, B2_cols, B2_vals, 20, _CH)
    pb1r = _prep(B1_rows, B1_cols, B1_vals, 2, _CHN)
    pb1c = _prep(B1_cols, B1_rows, B1_vals, 20, _CH)

    spmm_tri = _make_spmm(pb2c[0].shape[0], 10, _CH, _F)
    spmm_e_t = _make_spmm(pb2r[0].shape[0], 20, _CH, _F)
    spmm_n = _make_spmm(pb1r[0].shape[0], 2, _CHN, _F)
    spmm_e_n = _make_spmm(pb1c[0].shape[0], 20, _CH, _F)

    p0, d1, p2 = _mm3(xp, W0s[0], W1s[0], W2s[0])
    hw = None
    for i in range(2):
        t = spmm_tri(p2, *pb2c, z128)
        d2 = spmm_e_t(t, *pb2r, z128)
        nn = spmm_n(p0, *pb1r, z128)
        d0 = spmm_e_n(nn, *pb1c, z128)
        if i == 0:
            p0, d1, p2 = _relu_mm3(d0, d1, d2, W0s[1], W1s[1], W2s[1])
        else:
            hw = _relu_mm1(d0, d1, d2, jnp.pad(W0_L, ((0, 0), (0, 127))))
    res = spmm_n(hw, *pb1r, z128)
    return res[:10000, :1]


# speedup vs baseline: 2.1019x; 1.0013x over previous
import functools

import jax
import jax.numpy as jnp
from jax import lax
from jax.experimental import pallas as pl
from jax.experimental.pallas import tpu as pltpu
from jax.experimental.pallas import tpu_sc as plsc

_E = 160000
_F = 128
_CH = 8192
_CHN = 6144
_E_PAD = 20 * _CH
_T_PAD = 10 * _CH
_N_PAD = 2 * _CHN
_BLK = 1280
_TILE = 128


def _mm3_body(x_ref, w0_ref, w1_ref, w2_ref, o0, o1, o2):
    xb = x_ref[...]
    o0[...] = jnp.dot(xb, w0_ref[...], preferred_element_type=jnp.float32)
    o1[...] = jnp.dot(xb, w1_ref[...], preferred_element_type=jnp.float32)
    o2[...] = jnp.dot(xb, w2_ref[...], preferred_element_type=jnp.float32)


def _relu_mm3_body(a_ref, b_ref, c_ref, w0_ref, w1_ref, w2_ref, o0, o1, o2):
    h = jnp.maximum(a_ref[...] + b_ref[...] + c_ref[...], 0.0)
    o0[...] = jnp.dot(h, w0_ref[...], preferred_element_type=jnp.float32)
    o1[...] = jnp.dot(h, w1_ref[...], preferred_element_type=jnp.float32)
    o2[...] = jnp.dot(h, w2_ref[...], preferred_element_type=jnp.float32)


def _relu_mm1_body(a_ref, b_ref, c_ref, w_ref, o0):
    h = jnp.maximum(a_ref[...] + b_ref[...] + c_ref[...], 0.0)
    o0[...] = jnp.dot(h, w_ref[...], preferred_element_type=jnp.float32)


def _row_spec(width):
    return pl.BlockSpec((_BLK, width), lambda i: (i, 0))


def _w_spec(width):
    return pl.BlockSpec((_F, width), lambda i: (0, 0))


_G = _E_PAD // _BLK

_mm3 = pl.pallas_call(
    _mm3_body,
    grid=(_G,),
    in_specs=[_row_spec(_F), _w_spec(_F), _w_spec(_F), _w_spec(_F)],
    out_specs=[_row_spec(_F)] * 3,
    out_shape=[jax.ShapeDtypeStruct((_E_PAD, _F), jnp.float32)] * 3,
)

_relu_mm3 = pl.pallas_call(
    _relu_mm3_body,
    grid=(_G,),
    in_specs=[_row_spec(_F)] * 3 + [_w_spec(_F)] * 3,
    out_specs=[_row_spec(_F)] * 3,
    out_shape=[jax.ShapeDtypeStruct((_E_PAD, _F), jnp.float32)] * 3,
)

_relu_mm1 = pl.pallas_call(
    _relu_mm1_body,
    grid=(_G,),
    in_specs=[_row_spec(_F)] * 3 + [_w_spec(_F)],
    out_specs=_row_spec(_F),
    out_shape=jax.ShapeDtypeStruct((_E_PAD, _F), jnp.float32),
)


def _prep(tgt, srcidx, vals, nchunk, ch):
    nnz = tgt.shape[0]
    nnzp = nnz + _TILE * nchunk
    chunk_id = (tgt // ch).astype(jnp.int32)
    key = jnp.sort(chunk_id * 524288 + jnp.arange(nnz, dtype=jnp.int32))
    order = key % 524288
    tgt_s = tgt[order]
    src_s = srcidx[order]
    val_s = vals[order]
    edges = jnp.arange(nchunk + 1, dtype=jnp.int32) * 524288
    bounds = jnp.searchsorted(key, edges).astype(jnp.int32)
    cnt = bounds[1:] - bounds[:-1]
    tiles = (cnt + _TILE - 1) // _TILE
    padlen = tiles * _TILE
    pend = jnp.cumsum(padlen).astype(jnp.int32)
    pstart = (pend - padlen).astype(jnp.int32)
    j = jnp.arange(nnzp, dtype=jnp.int32)
    c_of = jnp.minimum((j[:, None] >= pend[None, :]).astype(jnp.int32).sum(axis=1),
                       nchunk - 1)
    within = j - pstart[c_of]
    valid = within < cnt[c_of]
    spos = jnp.clip(bounds[:-1][c_of] + within, 0, nnz - 1)
    psrc = jnp.where(valid, src_s[spos], 0).astype(jnp.int32)
    ptgt = jnp.where(valid, tgt_s[spos], c_of * ch).astype(jnp.int32)
    pval = jnp.where(valid, val_s[spos], 0.0).astype(jnp.float32)
    meta_t = jnp.broadcast_to(tiles[:, None], (nchunk, 16)).reshape(-1)
    meta_s = jnp.broadcast_to((pstart // _TILE)[:, None], (nchunk, 16)).reshape(-1)
    return psrc, ptgt, pval, meta_t, meta_s


@functools.lru_cache(maxsize=None)
def _make_spmm(nnzp, nchunk, ch, f):
    del nnzp
    out_rows = nchunk * ch
    sub_rows = ch // 16
    nzcopy = sub_rows // _TILE
    mesh = plsc.VectorSubcoreMesh(core_axis_name="c", subcore_axis_name="s")

    @functools.partial(
        pl.kernel,
        mesh=mesh,
        out_type=jax.ShapeDtypeStruct((out_rows, f), jnp.float32),
        scratch_types=[
            pltpu.VMEM((_TILE,), jnp.int32),
            pltpu.VMEM((_TILE,), jnp.int32),
            pltpu.VMEM((_TILE,), jnp.int32),
            pltpu.VMEM((_TILE,), jnp.int32),
            pltpu.VMEM((_TILE,), jnp.int32),
            pltpu.VMEM((_TILE,), jnp.int32),
            pltpu.VMEM((_TILE,), jnp.float32),
            pltpu.VMEM((_TILE,), jnp.float32),
            pltpu.VMEM((_TILE, f), jnp.float32),
            pltpu.VMEM((_TILE, f), jnp.float32),
            pltpu.VMEM((_TILE, f), jnp.float32),
            pltpu.VMEM((nchunk * 16,), jnp.int32),
            pltpu.VMEM((nchunk * 16,), jnp.int32),
            pltpu.VMEM_SHARED((ch, f), jnp.float32),
            pltpu.SemaphoreType.DMA,
            pltpu.SemaphoreType.DMA,
            pltpu.SemaphoreType.DMA,
            pltpu.SemaphoreType.DMA,
            pltpu.SemaphoreType.DMA,
            pltpu.SemaphoreType.DMA,
            pltpu.SemaphoreType.DMA,
        ],
    )
    def spmm(src_hbm, psrc_hbm, ptgt_hbm, pvalr_hbm, mt_hbm, ms_hbm, z_hbm,
             out_hbm, idx0, idx1, tgt0, tgt1, tl0, tl1, val0, val1, rows0,
             rows1, zero_v, mt_v, ms_v, acc_sh, sa0, sa1, sg0, sg1, ss0,
             ss1, sz):
        idx = (idx0, idx1)
        tgt = (tgt0, tgt1)
        tl = (tl0, tl1)
        val = (val0, val1)
        rows = (rows0, rows1)
        sa = (sa0, sa1)
        sg = (sg0, sg1)
        ss = (ss0, ss1)
        cid = lax.axis_index("c")
        sid = lax.axis_index("s")
        pltpu.sync_copy(mt_hbm, mt_v)
        pltpu.sync_copy(ms_hbm, ms_v)
        pltpu.sync_copy(z_hbm, zero_v)

        def chunk_body(ci, _):
            c = 2 * ci + cid
            for z in range(nzcopy):
                r0 = sid * sub_rows + z * _TILE
                pltpu.async_copy(zero_v, acc_sh.at[pl.ds(r0, _TILE), :], sz)
            for z in range(nzcopy):
                pltpu.make_async_copy(
                    zero_v, acc_sh.at[pl.ds(sid * sub_rows, _TILE), :], sz
                ).wait()
            plsc.subcore_barrier()
            t_c = mt_v[pl.ds(c * 16, 16)][0]
            s_c = ms_v[pl.ds(c * 16, 16)][0]
            my_tiles = (t_c - sid + 15) // 16
            base_l = c * ch

            def issue_in(i, b):
                off = (s_c + (sid + i * 16)) * _TILE
                pltpu.async_copy(psrc_hbm.at[pl.ds(off, _TILE)], idx[b], sa[b])
                pltpu.async_copy(ptgt_hbm.at[pl.ds(off, _TILE)], tgt[b], sa[b])
                pltpu.async_copy(pvalr_hbm.at[pl.ds(off, _TILE)], val[b], sa[b])

            @pl.when(my_tiles > 0)
            def _():
                issue_in(0, 0)

            def pair_body(g, _):
                for b in (0, 1):
                    i = g * 2 + b

                    @pl.when(i < my_tiles)
                    def _(b=b, i=i):
                        pltpu.make_async_copy(
                            psrc_hbm.at[pl.ds(0, _TILE)], idx[b], sa[b]).wait()
                        pltpu.make_async_copy(
                            ptgt_hbm.at[pl.ds(0, _TILE)], tgt[b], sa[b]).wait()
                        pltpu.make_async_copy(
                            pvalr_hbm.at[pl.ds(0, _TILE)], val[b], sa[b]).wait()

                        @pl.when(i >= 2)
                        def _():
                            pltpu.make_async_copy(
                                rows[b], acc_sh.at[tl[b]], ss[b]).wait()

                        gh = pltpu.async_copy(src_hbm.at[idx[b]], rows[b], sg[b])

                        @pl.when(i + 1 < my_tiles)
                        def _():
                            issue_in(i + 1, 1 - b)

                        def grp(gg, _):
                            s = pl.ds(gg * 16, 16)
                            tl[b][s] = tgt[b][s] - base_l
                            return 0

                        lax.fori_loop(0, _TILE // 16, grp, 0)
                        gh.wait()

                        def ent(gg, _):
                            vv = val[b][pl.ds(gg * 16, 16)]
                            for kk in range(16):
                                k = gg * 16 + kk
                                sp = lax.broadcast_in_dim(vv[kk], (16,), ())
                                for jj in range(f // 16):
                                    sl = pl.ds(jj * 16, 16)
                                    rows[b][k, sl] = rows[b][k, sl] * sp
                            return 0

                        lax.fori_loop(0, _TILE // 16, ent, 0)
                        pltpu.async_copy(
                            rows[b], acc_sh.at[tl[b]], ss[b], add=True)
                return 0

            lax.fori_loop(0, (my_tiles + 1) // 2, pair_body, 0)

            @pl.when(my_tiles >= 1)
            def _():
                pltpu.make_async_copy(rows[0], acc_sh.at[tl[0]], ss[0]).wait()

            @pl.when(my_tiles >= 2)
            def _():
                pltpu.make_async_copy(rows[1], acc_sh.at[tl[1]], ss[1]).wait()

            plsc.subcore_barrier()
            for z in range(nzcopy):
                r0 = sid * sub_rows + z * _TILE
                pltpu.async_copy(
                    acc_sh.at[pl.ds(r0, _TILE), :],
                    out_hbm.at[pl.ds(base_l + r0, _TILE), :], sz)
            for z in range(nzcopy):
                pltpu.make_async_copy(
                    acc_sh.at[pl.ds(sid * sub_rows, _TILE), :],
                    out_hbm.at[pl.ds(base_l, _TILE), :], sz).wait()
            return 0

        lax.fori_loop(0, nchunk // 2, chunk_body, 0)

    return spmm


def kernel(x, W0s, W1s, W2s, W0_L, B1_rows, B1_cols, B1_vals, B2_rows, B2_cols, B2_vals):
    xp = jnp.pad(x, ((0, _E_PAD - _E), (0, 0)))
    z128 = jnp.zeros((_TILE, _F), jnp.float32)

    pb2c = _prep(B2_cols, B2_rows, B2_vals, 10, _CH)

    spmm_tri = _make_spmm(pb2c[0].shape[0], 10, _CH, _F)

    p0, d1, p2 = _mm3(xp, W0s[0], W1s[0], W2s[0])
    hw = None
    pb2r = pb1r = pb1c = None
    spmm_e_t = spmm_n = spmm_e_n = None
    for i in range(2):
        t = spmm_tri(p2, *pb2c, z128)
        if i == 0:
            pb2r = _prep(B2_rows, B2_cols, B2_vals, 20, _CH)
            spmm_e_t = _make_spmm(pb2r[0].shape[0], 20, _CH, _F)
        d2 = spmm_e_t(t, *pb2r, z128)
        if i == 0:
            pb1r = _prep(B1_rows, B1_cols, B1_vals, 2, _CHN)
            spmm_n = _make_spmm(pb1r[0].shape[0], 2, _CHN, _F)
        nn = spmm_n(p0, *pb1r, z128)
        if i == 0:
            pb1c = _prep(B1_cols, B1_rows, B1_vals, 20, _CH)
            spmm_e_n = _make_spmm(pb1c[0].shape[0], 20, _CH, _F)
        d0 = spmm_e_n(nn, *pb1c, z128)
        if i == 0:
            p0, d1, p2 = _relu_mm3(d0, d1, d2, W0s[1], W1s[1], W2s[1])
        else:
            hw = _relu_mm1(d0, d1, d2, jnp.pad(W0_L, ((0, 0), (0, 127))))
    res = spmm_n(hw, *pb1r, z128)
    return res[:10000, :1]
